# baseline (speedup 1.0000x reference)
import functools
import jax
import jax.numpy as jnp
from jax.experimental import pallas as pl

BB = 128
JP = 32


def _prep_graph(adj, A2, M, W):
    A = adj + A2
    As = (A.T + A) * 0.5
    d = jnp.diagonal(As)
    J = adj.shape[0]
    Aoff = As - jnp.diag(d)
    Aoff_p = jnp.zeros((JP, JP), jnp.float32).at[:J, :J].set(Aoff)
    T = jnp.kron(jnp.eye(128 // JP, dtype=jnp.float32), Aoff_p)
    dcoef = jnp.zeros((JP, M.shape[1]), jnp.float32).at[:J].set(
        d[:, None] * M)
    Mp = jnp.zeros((JP, M.shape[1]), jnp.float32).at[:J].set(M)
    Wcat = jnp.concatenate([W[0], W[1]], axis=1)
    return T.astype(jnp.bfloat16), dcoef, Mp, Wcat.astype(jnp.bfloat16)


def _mgconv_padded(xp2, Wcat_ref, T_ref, dcoef_ref, Mp_ref, b_ref):
    R, F = xp2.shape
    Fo = Wcat_ref.shape[-1] // 2
    xm = xp2.astype(jnp.bfloat16)
    h01 = jnp.dot(xm, Wcat_ref[...], preferred_element_type=jnp.float32)
    h0 = h01[:, :Fo]
    h1 = h01[:, Fo:]
    z = h1.reshape(R // JP, JP, Fo) * Mp_ref[...][None]
    C = R // 128
    zc = z.reshape(C, 128, Fo).astype(jnp.bfloat16)
    Tc = jnp.broadcast_to(T_ref[...][None], (C, 128, 128))
    offc = jax.lax.dot_general(Tc, zc, (((2,), (1,)), ((0,), (0,))),
                               preferred_element_type=jnp.float32)
    off = offc.reshape(R // JP, JP, Fo)
    diag = h0.reshape(R // JP, JP, Fo) * dcoef_ref[...][None]
    return diag + off + b_ref[...][None, None, :]


def _bn_relu3(h, acc_ref, g_ref, be_ref, bprev_ref, n, npad):
    b = bprev_ref[...]
    s0 = acc_ref[0, :] - npad * b
    s1 = acc_ref[1, :] - npad * b * b
    mean = s0 / n
    var = s1 / n - mean * mean
    inv = jax.lax.rsqrt(var + 1e-5)
    scale = g_ref[...] * inv
    shift = be_ref[...] - mean * scale
    return jnp.maximum(h * scale[None, None, :] + shift[None, None, :], 0.0)


def _acc_plain(acc_ref, out):
    i = pl.program_id(0)

    @pl.when(i == 0)
    def _():
        acc_ref[...] = jnp.zeros_like(acc_ref)

    acc_ref[0, :] += jnp.sum(out, axis=(0, 1))
    acc_ref[1, :] += jnp.sum(out * out, axis=(0, 1))


def _pad_joints(xb):
    Bb, J, F = xb.shape
    return jnp.concatenate(
        [xb, jnp.zeros((Bb, JP - J, F), xb.dtype)], axis=1)


def _p1_kernel(x_ref, Wc_ref, T_ref, dcoef_ref, Mp_ref, b_ref, h_ref,
               acc_ref):
    xp = _pad_joints(x_ref[...])
    out = _mgconv_padded(xp.reshape(-1, xp.shape[-1]), Wc_ref, T_ref,
                         dcoef_ref, Mp_ref, b_ref)
    h_ref[...] = out.astype(h_ref.dtype)
    _acc_plain(acc_ref, out)


def _p2_kernel(h_ref, acc1_ref, g_ref, be_ref, b1_ref, Wc_ref, T_ref,
               dcoef_ref, Mp_ref, b_ref, h2_ref, acc2_ref, *, n, npad):
    a = _bn_relu3(h_ref[...].astype(jnp.float32), acc1_ref, g_ref, be_ref,
                  b1_ref, n, npad)
    out = _mgconv_padded(a.reshape(-1, a.shape[-1]), Wc_ref, T_ref,
                         dcoef_ref, Mp_ref, b_ref)
    h2_ref[...] = out.astype(h2_ref.dtype)
    _acc_plain(acc2_ref, out)


def _p3_kernel(x_ref, h2_ref, acc2_ref, g_ref, be_ref, b2_ref, Wco_ref,
               To_ref, dco_ref, Mop_ref, bo_ref, out_ref, *, n, npad, J):
    a = _bn_relu3(h2_ref[...].astype(jnp.float32), acc2_ref, g_ref, be_ref,
                  b2_ref, n, npad)
    h = _pad_joints(x_ref[...]) + a
    o = _mgconv_padded(h.reshape(-1, h.shape[-1]), Wco_ref, To_ref,
                       dco_ref, Mop_ref, bo_ref)
    out_ref[...] = o[:, :J, :]


def _full(shape):
    rank = len(shape)
    return pl.BlockSpec(shape, lambda i, _r=rank: (0,) * _r)


def kernel(x, adj, W1, M1, A2_1, b1, g1, be1, W2, M2, A2_2, b2, g2, be2,
           Wo, Mo, A2o, bo, interpret=False):
    B, J, F = x.shape
    Fo = Wo.shape[-1]
    n = float(B * J)
    npad = float(B * (JP - J))
    T1, dc1, Mp1, Wc1 = _prep_graph(adj, A2_1, M1, W1)
    T2, dc2, Mp2, Wc2 = _prep_graph(adj, A2_2, M2, W2)
    To, dco, Mpo, Wco = _prep_graph(adj, A2o, Mo, Wo)

    grid = (B // BB,)
    xblk = pl.BlockSpec((BB, J, F), lambda i: (i, 0, 0))
    pblk = pl.BlockSpec((BB, JP, F), lambda i: (i, 0, 0))
    acc_spec = pl.BlockSpec((2, F), lambda i: (0, 0))
    hp_sds = jax.ShapeDtypeStruct((B, JP, F), jnp.bfloat16)
    acc_sds = jax.ShapeDtypeStruct((2, F), jnp.float32)

    h1p, acc1 = pl.pallas_call(
        _p1_kernel,
        grid=grid,
        in_specs=[xblk, _full(Wc1.shape), _full(T1.shape), _full(dc1.shape),
                  _full(Mp1.shape), _full(b1.shape)],
        out_specs=[pblk, acc_spec],
        out_shape=[hp_sds, acc_sds],
        interpret=interpret,
    )(x, Wc1, T1, dc1, Mp1, b1)

    h2p, acc2 = pl.pallas_call(
        functools.partial(_p2_kernel, n=n, npad=npad),
        grid=grid,
        in_specs=[pblk, acc_spec, _full(g1.shape), _full(be1.shape),
                  _full(b1.shape), _full(Wc2.shape), _full(T2.shape),
                  _full(dc2.shape), _full(Mp2.shape), _full(b2.shape)],
        out_specs=[pblk, acc_spec],
        out_shape=[hp_sds, acc_sds],
        interpret=interpret,
    )(h1p, acc1, g1, be1, b1, Wc2, T2, dc2, Mp2, b2)

    out = pl.pallas_call(
        functools.partial(_p3_kernel, n=n, npad=npad, J=J),
        grid=grid,
        in_specs=[xblk, pblk, acc_spec, _full(g2.shape), _full(be2.shape),
                  _full(b2.shape), _full(Wco.shape), _full(To.shape),
                  _full(dco.shape), _full(Mpo.shape), _full(bo.shape)],
        out_specs=pl.BlockSpec((BB, J, Fo), lambda i: (i, 0, 0)),
        out_shape=jax.ShapeDtypeStruct((B, J, Fo), jnp.float32),
        interpret=interpret,
    )(x, h2p, acc2, g2, be2, b2, Wco, To, dco, Mpo, bo)
    return out


# speedup vs baseline: 1.0104x; 1.0104x over previous
import functools
import jax
import jax.numpy as jnp
from jax.experimental import pallas as pl

BB = 128
JP = 32


def _prep_graph(adj, A2, M):
    A = adj + A2
    As = (A.T + A) * 0.5
    d = jnp.diagonal(As)
    J = adj.shape[0]
    Aoff = As - jnp.diag(d)
    Aoff_p = jnp.zeros((JP, JP), jnp.float32).at[:J, :J].set(Aoff)
    T = jnp.kron(jnp.eye(128 // JP, dtype=jnp.float32), Aoff_p)
    dcoef = jnp.zeros((JP, M.shape[1]), jnp.float32).at[:J].set(
        d[:, None] * M)
    Mp = jnp.zeros((JP, M.shape[1]), jnp.float32).at[:J].set(M)
    return T.astype(jnp.bfloat16), dcoef, Mp


def _mgconv_padded(xp2, W_ref, T_ref, dcoef_ref, Mp_ref, b_ref):
    R, F = xp2.shape
    Fo = W_ref.shape[-1]
    xm = xp2.astype(jnp.bfloat16)
    h0 = jnp.dot(xm, W_ref[0].astype(jnp.bfloat16),
                 preferred_element_type=jnp.float32)
    h1 = jnp.dot(xm, W_ref[1].astype(jnp.bfloat16),
                 preferred_element_type=jnp.float32)
    z = h1.reshape(R // JP, JP, Fo) * Mp_ref[...][None]
    C = R // 128
    zc = z.reshape(C, 128, Fo).astype(jnp.bfloat16)
    Tc = jnp.broadcast_to(T_ref[...][None], (C, 128, 128))
    offc = jax.lax.dot_general(Tc, zc, (((2,), (1,)), ((0,), (0,))),
                               preferred_element_type=jnp.float32)
    off = offc.reshape(R // JP, JP, Fo)
    diag = h0.reshape(R // JP, JP, Fo) * dcoef_ref[...][None]
    return diag + off + b_ref[...][None, None, :]


def _bn_relu3(h, acc_ref, g_ref, be_ref, bprev_ref, n, npad):
    b = bprev_ref[...]
    s0 = acc_ref[0, :] - npad * b
    s1 = acc_ref[1, :] - npad * b * b
    mean = s0 / n
    var = s1 / n - mean * mean
    inv = jax.lax.rsqrt(var + 1e-5)
    scale = g_ref[...] * inv
    shift = be_ref[...] - mean * scale
    return jnp.maximum(h * scale[None, None, :] + shift[None, None, :], 0.0)


def _acc_plain(acc_ref, out):
    i = pl.program_id(0)

    @pl.when(i == 0)
    def _():
        acc_ref[...] = jnp.zeros_like(acc_ref)

    acc_ref[0, :] += jnp.sum(out, axis=(0, 1))
    acc_ref[1, :] += jnp.sum(out * out, axis=(0, 1))


def _pad_joints(xb):
    Bb, J, F = xb.shape
    return jnp.concatenate(
        [xb, jnp.zeros((Bb, JP - J, F), xb.dtype)], axis=1)


def _p1_kernel(x_ref, W_ref, T_ref, dcoef_ref, Mp_ref, b_ref, h_ref,
               acc_ref):
    xp = _pad_joints(x_ref[...])
    out = _mgconv_padded(xp.reshape(-1, xp.shape[-1]), W_ref, T_ref,
                         dcoef_ref, Mp_ref, b_ref)
    h_ref[...] = out.astype(h_ref.dtype)
    _acc_plain(acc_ref, out)


def _p2_kernel(h_ref, acc1_ref, g_ref, be_ref, b1_ref, W_ref, T_ref,
               dcoef_ref, Mp_ref, b_ref, h2_ref, acc2_ref, *, n, npad):
    a = _bn_relu3(h_ref[...].astype(jnp.float32), acc1_ref, g_ref, be_ref,
                  b1_ref, n, npad)
    out = _mgconv_padded(a.reshape(-1, a.shape[-1]), W_ref, T_ref,
                         dcoef_ref, Mp_ref, b_ref)
    h2_ref[...] = out.astype(h2_ref.dtype)
    _acc_plain(acc2_ref, out)


def _p3_kernel(x_ref, h2_ref, acc2_ref, g_ref, be_ref, b2_ref, Wo_ref,
               To_ref, dco_ref, Mop_ref, bo_ref, out_ref, *, n, npad, J):
    a = _bn_relu3(h2_ref[...].astype(jnp.float32), acc2_ref, g_ref, be_ref,
                  b2_ref, n, npad)
    h = _pad_joints(x_ref[...]) + a
    o = _mgconv_padded(h.reshape(-1, h.shape[-1]), Wo_ref, To_ref,
                       dco_ref, Mop_ref, bo_ref)
    out_ref[...] = o[:, :J, :]


def _full(shape):
    rank = len(shape)
    return pl.BlockSpec(shape, lambda i, _r=rank: (0,) * _r)


def kernel(x, adj, W1, M1, A2_1, b1, g1, be1, W2, M2, A2_2, b2, g2, be2,
           Wo, Mo, A2o, bo, interpret=False):
    B, J, F = x.shape
    Fo = Wo.shape[-1]
    n = float(B * J)
    npad = float(B * (JP - J))
    T1, dc1, Mp1 = _prep_graph(adj, A2_1, M1)
    T2, dc2, Mp2 = _prep_graph(adj, A2_2, M2)
    To, dco, Mpo = _prep_graph(adj, A2o, Mo)

    grid = (B // BB,)
    xblk = pl.BlockSpec((BB, J, F), lambda i: (i, 0, 0))
    pblk = pl.BlockSpec((BB, JP, F), lambda i: (i, 0, 0))
    acc_spec = pl.BlockSpec((2, F), lambda i: (0, 0))
    hp_sds = jax.ShapeDtypeStruct((B, JP, F), jnp.bfloat16)
    acc_sds = jax.ShapeDtypeStruct((2, F), jnp.float32)

    h1p, acc1 = pl.pallas_call(
        _p1_kernel,
        grid=grid,
        in_specs=[xblk, _full(W1.shape), _full(T1.shape), _full(dc1.shape),
                  _full(Mp1.shape), _full(b1.shape)],
        out_specs=[pblk, acc_spec],
        out_shape=[hp_sds, acc_sds],
        interpret=interpret,
    )(x, W1, T1, dc1, Mp1, b1)

    h2p, acc2 = pl.pallas_call(
        functools.partial(_p2_kernel, n=n, npad=npad),
        grid=grid,
        in_specs=[pblk, acc_spec, _full(g1.shape), _full(be1.shape),
                  _full(b1.shape), _full(W2.shape), _full(T2.shape),
                  _full(dc2.shape), _full(Mp2.shape), _full(b2.shape)],
        out_specs=[pblk, acc_spec],
        out_shape=[hp_sds, acc_sds],
        interpret=interpret,
    )(h1p, acc1, g1, be1, b1, W2, T2, dc2, Mp2, b2)

    out = pl.pallas_call(
        functools.partial(_p3_kernel, n=n, npad=npad, J=J),
        grid=grid,
        in_specs=[xblk, pblk, acc_spec, _full(g2.shape), _full(be2.shape),
                  _full(b2.shape), _full(Wo.shape), _full(To.shape),
                  _full(dco.shape), _full(Mpo.shape), _full(bo.shape)],
        out_specs=pl.BlockSpec((BB, J, Fo), lambda i: (i, 0, 0)),
        out_shape=jax.ShapeDtypeStruct((B, J, Fo), jnp.float32),
        interpret=interpret,
    )(x, h2p, acc2, g2, be2, b2, Wo, To, dco, Mpo, bo)
    return out


# speedup vs baseline: 1.0711x; 1.0600x over previous
import functools
import jax
import jax.numpy as jnp
from jax.experimental import pallas as pl

BB = 256
JP = 32


def _prep_graph(adj, A2, M, dtype):
    A = adj + A2
    As = (A.T + A) * 0.5
    d = jnp.diagonal(As)
    J = adj.shape[0]
    Aoff = As - jnp.diag(d)
    Aoff_p = jnp.zeros((JP, JP), dtype).at[:J, :J].set(Aoff)
    T = jnp.kron(jnp.eye(128 // JP, dtype=dtype), Aoff_p)
    dcoef = jnp.zeros((JP, M.shape[1]), jnp.float32).at[:J].set(
        d[:, None] * M)
    Mp = jnp.zeros((JP, M.shape[1]), jnp.float32).at[:J].set(M)
    return T, dcoef, Mp


def _mgconv_padded(xp2, W_ref, T_ref, dcoef_ref, Mp_ref, b_ref, mm_dtype):
    R, F = xp2.shape
    Fo = W_ref.shape[-1]
    xm = xp2.astype(mm_dtype)
    h0 = jnp.dot(xm, W_ref[0].astype(mm_dtype),
                 preferred_element_type=jnp.float32)
    h1 = jnp.dot(xm, W_ref[1].astype(mm_dtype),
                 preferred_element_type=jnp.float32)
    z = h1.reshape(R // JP, JP, Fo) * Mp_ref[...][None]
    C = R // 128
    zc = z.reshape(C, 128, Fo).astype(mm_dtype)
    Tc = jnp.broadcast_to(T_ref[...].astype(mm_dtype)[None], (C, 128, 128))
    offc = jax.lax.dot_general(Tc, zc, (((2,), (1,)), ((0,), (0,))),
                               preferred_element_type=jnp.float32)
    off = offc.reshape(R // JP, JP, Fo)
    diag = h0.reshape(R // JP, JP, Fo) * dcoef_ref[...][None]
    return diag + off + b_ref[...][None, None, :]


def _bn_relu3(h, acc_ref, g_ref, be_ref, n):
    mean = acc_ref[0, :] / n
    var = acc_ref[1, :] / n - mean * mean
    inv = jax.lax.rsqrt(var + 1e-5)
    scale = g_ref[...] * inv
    shift = be_ref[...] - mean * scale
    return jnp.maximum(h * scale[None, None, :] + shift[None, None, :], 0.0)


def _acc_masked(acc_ref, out, J):
    i = pl.program_id(0)
    jidx = jax.lax.broadcasted_iota(jnp.int32, out.shape, 1)
    o = jnp.where(jidx < J, out, 0.0)

    @pl.when(i == 0)
    def _():
        acc_ref[...] = jnp.zeros_like(acc_ref)

    acc_ref[0, :] += jnp.sum(o, axis=(0, 1))
    acc_ref[1, :] += jnp.sum(o * o, axis=(0, 1))


def _pad_joints(xb):
    Bb, J, F = xb.shape
    return jnp.concatenate(
        [xb, jnp.zeros((Bb, JP - J, F), xb.dtype)], axis=1)


def _p1_kernel(x_ref, W_ref, T_ref, dcoef_ref, Mp_ref, b_ref, h_ref,
               acc_ref, *, J):
    xp = _pad_joints(x_ref[...])
    out = _mgconv_padded(xp.reshape(-1, xp.shape[-1]), W_ref, T_ref,
                         dcoef_ref, Mp_ref, b_ref, jnp.bfloat16)
    h_ref[...] = out.astype(h_ref.dtype)
    _acc_masked(acc_ref, out, J)


def _p2_kernel(h_ref, acc1_ref, g_ref, be_ref, W_ref, T_ref, dcoef_ref,
               Mp_ref, b_ref, h2_ref, acc2_ref, *, n, J):
    a = _bn_relu3(h_ref[...].astype(jnp.float32), acc1_ref, g_ref, be_ref, n)
    out = _mgconv_padded(a.reshape(-1, a.shape[-1]), W_ref, T_ref,
                         dcoef_ref, Mp_ref, b_ref, jnp.bfloat16)
    h2_ref[...] = out.astype(h2_ref.dtype)
    _acc_masked(acc2_ref, out, J)


def _p3_kernel(x_ref, h2_ref, acc2_ref, g_ref, be_ref, Wo_ref, To_ref,
               dco_ref, Mop_ref, bo_ref, out_ref, *, n, J):
    a = _bn_relu3(h2_ref[...].astype(jnp.float32), acc2_ref, g_ref, be_ref, n)
    h = _pad_joints(x_ref[...]) + a
    o = _mgconv_padded(h.reshape(-1, h.shape[-1]), Wo_ref, To_ref,
                       dco_ref, Mop_ref, bo_ref, jnp.float32)
    out_ref[...] = o[:, :J, :]


def _full(shape):
    rank = len(shape)
    return pl.BlockSpec(shape, lambda i, _r=rank: (0,) * _r)


def kernel(x, adj, W1, M1, A2_1, b1, g1, be1, W2, M2, A2_2, b2, g2, be2,
           Wo, Mo, A2o, bo, interpret=False):
    B, J, F = x.shape
    Fo = Wo.shape[-1]
    n = float(B * J)
    T1, dc1, Mp1 = _prep_graph(adj, A2_1, M1, jnp.float32)
    T2, dc2, Mp2 = _prep_graph(adj, A2_2, M2, jnp.float32)
    To, dco, Mpo = _prep_graph(adj, A2o, Mo, jnp.float32)

    grid = (B // BB,)
    xblk = pl.BlockSpec((BB, J, F), lambda i: (i, 0, 0))
    pblk = pl.BlockSpec((BB, JP, F), lambda i: (i, 0, 0))
    acc_spec = pl.BlockSpec((2, F), lambda i: (0, 0))
    hp_sds = jax.ShapeDtypeStruct((B, JP, F), jnp.bfloat16)
    acc_sds = jax.ShapeDtypeStruct((2, F), jnp.float32)

    h1p, acc1 = pl.pallas_call(
        functools.partial(_p1_kernel, J=J),
        grid=grid,
        in_specs=[xblk, _full(W1.shape), _full(T1.shape), _full(dc1.shape),
                  _full(Mp1.shape), _full(b1.shape)],
        out_specs=[pblk, acc_spec],
        out_shape=[hp_sds, acc_sds],
        interpret=interpret,
    )(x, W1, T1, dc1, Mp1, b1)

    h2p, acc2 = pl.pallas_call(
        functools.partial(_p2_kernel, n=n, J=J),
        grid=grid,
        in_specs=[pblk, acc_spec, _full(g1.shape), _full(be1.shape),
                  _full(W2.shape), _full(T2.shape), _full(dc2.shape),
                  _full(Mp2.shape), _full(b2.shape)],
        out_specs=[pblk, acc_spec],
        out_shape=[hp_sds, acc_sds],
        interpret=interpret,
    )(h1p, acc1, g1, be1, W2, T2, dc2, Mp2, b2)

    out = pl.pallas_call(
        functools.partial(_p3_kernel, n=n, J=J),
        grid=grid,
        in_specs=[xblk, pblk, acc_spec, _full(g2.shape), _full(be2.shape),
                  _full(Wo.shape), _full(To.shape), _full(dco.shape),
                  _full(Mpo.shape), _full(bo.shape)],
        out_specs=pl.BlockSpec((BB, J, Fo), lambda i: (i, 0, 0)),
        out_shape=jax.ShapeDtypeStruct((B, J, Fo), jnp.float32),
        interpret=interpret,
    )(x, h2p, acc2, g2, be2, Wo, To, dco, Mpo, bo)
    return out
